# baseline (speedup 1.0000x reference)
import jax
import jax.numpy as jnp
from jax.experimental import pallas as pl
from jax.experimental.pallas import tpu as pltpu

_K = 15
_INV_TEMP = 10.0
_EPS = 1e-12
_NEG = -3.0e38


def _pool(src, dst, row0, br):
    x = jnp.mean(src[...], axis=1)
    nrm = jnp.sqrt(jnp.sum(x * x, axis=1, keepdims=True))
    dst[pl.ds(row0, br), :] = x / jnp.maximum(nrm, _EPS)


def _fused_kernel(a_ref, m_ref, out_ref, an_s, mn_s, sim_s, thr_s):
    i = pl.program_id(0)
    nb = pl.num_programs(0) // 2
    br = a_ref.shape[0]
    b = an_s.shape[0]
    dn = (((1,), (1,)), ((), ()))

    @pl.when(i < nb)
    def _phase_anchor():
        _pool(a_ref, an_s, i * br, br)

    @pl.when(i >= nb)
    def _phase_modality():
        c = i - nb
        _pool(m_ref, mn_s, c * br, br)

        ar = an_s[pl.ds(c * br, br), :]
        sim = jax.lax.dot_general(ar, an_s[...], dn,
                                  preferred_element_type=jnp.float32)
        row_ids = c * br + jax.lax.broadcasted_iota(jnp.int32, (br, b), 0)
        col_ids = jax.lax.broadcasted_iota(jnp.int32, (br, b), 1)
        sim = jnp.where(row_ids == col_ids, _NEG, sim)
        sim_s[pl.ds(c * br, br), :] = sim
        work = sim
        for _ in range(_K - 1):
            mx = jnp.max(work, axis=1, keepdims=True)
            work = jnp.where(work >= mx, _NEG, work)
        thr_s[pl.ds(c * br, br), :] = jnp.max(work, axis=1, keepdims=True)

    @pl.when(i == 2 * nb - 1)
    def _loss():
        rb = 256
        acc = jnp.zeros((1, 1), jnp.float32)
        for c in range(b // rb):
            mr = mn_s[pl.ds(c * rb, rb), :]
            sim_m = jax.lax.dot_general(mr, mn_s[...], dn,
                                        preferred_element_type=jnp.float32)
            sim_m = sim_m * _INV_TEMP
            e = jnp.exp(sim_m - _INV_TEMP)
            row_ids = c * rb + jax.lax.broadcasted_iota(jnp.int32, (rb, b), 0)
            col_ids = jax.lax.broadcasted_iota(jnp.int32, (rb, b), 1)
            e = jnp.where(row_ids == col_ids, 0.0, e)
            pos = sim_s[pl.ds(c * rb, rb), :] >= thr_s[pl.ds(c * rb, rb), :]
            pos_sum = jnp.sum(jnp.where(pos, e, 0.0), axis=1) + 1e-8
            all_sum = jnp.sum(e, axis=1) + 1e-8
            contrib = jnp.sum(jnp.log(pos_sum) - jnp.log(all_sum))
            acc += jnp.reshape(-contrib / b, (1, 1))
        out_ref[...] = acc


def kernel(anchor, modality):
    B, S, D = anchor.shape
    BR = 32
    nb = B // BR
    loss = pl.pallas_call(
        _fused_kernel,
        grid=(2 * nb,),
        in_specs=[
            pl.BlockSpec((BR, S, D), lambda i: (jnp.minimum(i, nb - 1), 0, 0)),
            pl.BlockSpec((BR, S, D), lambda i: (jnp.maximum(i - nb, 0), 0, 0)),
        ],
        out_specs=pl.BlockSpec((1, 1), lambda i: (0, 0)),
        out_shape=jax.ShapeDtypeStruct((1, 1), jnp.float32),
        scratch_shapes=[
            pltpu.VMEM((B, D), jnp.float32),
            pltpu.VMEM((B, D), jnp.float32),
            pltpu.VMEM((B, B), jnp.float32),
            pltpu.VMEM((B, 1), jnp.float32),
        ],
    )(anchor, modality)
    return loss[0, 0]


# speedup vs baseline: 1.3818x; 1.3818x over previous
import jax
import jax.numpy as jnp
from jax.experimental import pallas as pl
from jax.experimental.pallas import tpu as pltpu

_K = 15
_INV_TEMP = 10.0
_EPS = 1e-12
_NEG = -3.0e38


def _fused_kernel(a_ref, m_ref, out_ref, an_s, mn_s):
    i = pl.program_id(0)
    n = pl.num_programs(0)
    br = a_ref.shape[0]
    for src, dst in ((a_ref, an_s), (m_ref, mn_s)):
        x = jnp.mean(src[...], axis=1)
        nrm = jnp.sqrt(jnp.sum(x * x, axis=1, keepdims=True))
        dst[pl.ds(i * br, br), :] = x / jnp.maximum(nrm, _EPS)

    @pl.when(i == n - 1)
    def _loss():
        b = an_s.shape[0]
        rb = 256
        an = an_s[...]
        mn = mn_s[...]
        dn = (((1,), (1,)), ((), ()))
        acc = jnp.zeros((1, 1), jnp.float32)
        for c in range(b // rb):
            ar = an_s[pl.ds(c * rb, rb), :]
            sim_a = jax.lax.dot_general(ar, an, dn,
                                        preferred_element_type=jnp.float32)
            row_ids = c * rb + jax.lax.broadcasted_iota(jnp.int32, (rb, b), 0)
            col_ids = jax.lax.broadcasted_iota(jnp.int32, (rb, b), 1)
            is_diag = row_ids == col_ids
            sim_a = jnp.where(is_diag, _NEG, sim_a)

            work = sim_a
            for _ in range(_K - 1):
                mx = jnp.max(work, axis=1, keepdims=True)
                work = jnp.where(work >= mx, _NEG, work)
            thr = jnp.max(work, axis=1, keepdims=True)
            pos = sim_a >= thr

            mr = mn_s[pl.ds(c * rb, rb), :]
            sim_m = jax.lax.dot_general(mr, mn, dn,
                                        preferred_element_type=jnp.float32)
            sim_m = sim_m * _INV_TEMP
            e = jnp.exp(sim_m - _INV_TEMP)
            e = jnp.where(is_diag, 0.0, e)
            pos_sum = jnp.sum(jnp.where(pos, e, 0.0), axis=1) + 1e-8
            all_sum = jnp.sum(e, axis=1) + 1e-8
            contrib = jnp.sum(jnp.log(pos_sum) - jnp.log(all_sum))
            acc += jnp.reshape(-contrib / b, (1, 1))
        out_ref[...] = acc


def kernel(anchor, modality):
    B, S, D = anchor.shape
    BR = 32
    loss = pl.pallas_call(
        _fused_kernel,
        grid=(B // BR,),
        in_specs=[
            pl.BlockSpec((BR, S, D), lambda i: (i, 0, 0)),
            pl.BlockSpec((BR, S, D), lambda i: (i, 0, 0)),
        ],
        out_specs=pl.BlockSpec((1, 1), lambda i: (0, 0)),
        out_shape=jax.ShapeDtypeStruct((1, 1), jnp.float32),
        scratch_shapes=[
            pltpu.VMEM((B, D), jnp.float32),
            pltpu.VMEM((B, D), jnp.float32),
        ],
    )(anchor, modality)
    return loss[0, 0]
